# baseline (speedup 1.0000x reference)
import functools

import jax
import jax.numpy as jnp
from jax import lax
from jax.experimental import pallas as pl
from jax.experimental.pallas import tpu as pltpu
from jax.experimental.pallas import tpu_sc as plsc


def _argmin_body(z_ref, cb_ref, z2_ref, c2_ref, idx_ref):
    zb = z_ref[...]
    cb = cb_ref[...]
    z2 = z2_ref[...].reshape(1, -1)
    c2 = c2_ref[...].reshape(-1, 1)
    s = lax.dot_general(cb, zb, (((1,), (1,)), ((), ())),
                        preferred_element_type=jnp.float32)
    sq = (z2 + c2) - 2.0 * s
    d = jnp.sqrt(jnp.maximum(sq, 0.0))
    m = jnp.min(d, axis=0, keepdims=True)
    K = d.shape[0]
    j = lax.broadcasted_iota(jnp.int32, d.shape, 0)
    idx = jnp.min(jnp.where(d == m, j, K), axis=0).astype(jnp.int32)
    idx_ref[0, 0, :] = idx


def _compute_indices(zf, codebook, block_m):
    M, D = zf.shape
    K = codebook.shape[0]
    nb = M // block_m
    z2 = jnp.sum(zf * zf, axis=-1, keepdims=True).reshape(nb, 1, block_m)
    c2 = jnp.sum(codebook * codebook, axis=-1).reshape(K, 1)
    idx3 = pl.pallas_call(
        _argmin_body,
        grid=(nb,),
        in_specs=[
            pl.BlockSpec((block_m, D), lambda i: (i, 0)),
            pl.BlockSpec((K, D), lambda i: (0, 0)),
            pl.BlockSpec((1, 1, block_m), lambda i: (i, 0, 0)),
            pl.BlockSpec((K, 1), lambda i: (0, 0)),
        ],
        out_specs=pl.BlockSpec((1, 1, block_m), lambda i: (i, 0, 0)),
        out_shape=jax.ShapeDtypeStruct((nb, 1, block_m), jnp.int32),
    )(zf, codebook, z2, c2)
    return idx3.reshape(M)


def _gather_rows(idx, codebook, n_rows, d):
    info = plsc.get_sparse_core_info()
    nc, ns = info.num_cores, info.num_subcores
    nw = nc * ns
    k = codebook.shape[0]
    b_per_w = n_rows // nw
    ch = 32
    nch = b_per_w // ch
    mesh = plsc.VectorSubcoreMesh(core_axis_name="c", subcore_axis_name="s")

    @functools.partial(
        pl.kernel,
        mesh=mesh,
        out_type=jax.ShapeDtypeStruct((n_rows, d), jnp.float32),
        scratch_types=[
            pltpu.VMEM((b_per_w,), jnp.int32),
            pltpu.VMEM((ch, d), jnp.float32),
            pltpu.VMEM((ch, d), jnp.float32),
            pltpu.SemaphoreType.DMA,
            pltpu.SemaphoreType.DMA,
            pltpu.SemaphoreType.DMA,
            pltpu.SemaphoreType.DMA,
        ],
    )
    def gather_kernel(idx_hbm, cb_hbm, out_hbm, idx_v, buf0, buf1,
                      gs0, gs1, ws0, ws1):
        wid = lax.axis_index("s") * nc + lax.axis_index("c")
        base = wid * b_per_w
        bufs = (buf0, buf1)
        gsems = (gs0, gs1)
        wsems = (ws0, ws1)
        pltpu.sync_copy(idx_hbm.at[pl.ds(base, b_per_w)], idx_v)

        gathers = [pltpu.async_copy(
            cb_hbm.at[idx_v.at[pl.ds(0, ch)]], bufs[0], gsems[0])]
        writes = []
        for c in range(nch):
            if c + 1 < nch:
                if c >= 1:
                    writes[c - 1].wait()
                gathers.append(pltpu.async_copy(
                    cb_hbm.at[idx_v.at[pl.ds((c + 1) * ch, ch)]],
                    bufs[(c + 1) % 2], gsems[(c + 1) % 2]))
            gathers[c].wait()
            writes.append(pltpu.async_copy(
                bufs[c % 2], out_hbm.at[pl.ds(base + c * ch, ch)],
                wsems[c % 2]))
        writes[nch - 2].wait()
        writes[nch - 1].wait()

    return gather_kernel(idx, codebook)


def kernel(z, codebook):
    b, t, d = z.shape
    m = b * t
    h = m // 2
    zf = z.reshape(m, d)
    idx_a = _compute_indices(zf[:h], codebook, block_m=512)
    q_a = _gather_rows(idx_a, codebook, h, d)
    idx_b = _compute_indices(zf[h:], codebook, block_m=512)
    q_b = _gather_rows(idx_b, codebook, h, d)
    quant = jnp.concatenate([q_a, q_b], axis=0)
    idx = jnp.concatenate([idx_a, idx_b], axis=0)
    return quant.reshape(b, t, d), idx.reshape(b, t)


# speedup vs baseline: 1.3238x; 1.3238x over previous
import functools

import jax
import jax.numpy as jnp
from jax import lax
from jax.experimental import pallas as pl
from jax.experimental.pallas import tpu as pltpu
from jax.experimental.pallas import tpu_sc as plsc


def _argmin_body(z_ref, cb_hbm, z2_ref, c2_ref, idx_ref, cb_v, sem):
    @pl.when(pl.program_id(0) == 0)
    def _stage():
        cp = pltpu.make_async_copy(cb_hbm, cb_v, sem)
        cp.start()
        cp.wait()

    zb = z_ref[...]
    cb = cb_v[...]
    z2 = z2_ref[...].reshape(1, -1)
    c2 = c2_ref[...].reshape(-1, 1)
    s = lax.dot_general(cb, zb, (((1,), (1,)), ((), ())),
                        preferred_element_type=jnp.float32)
    sq = (z2 + c2) - 2.0 * s
    d = jnp.sqrt(jnp.maximum(sq, 0.0))
    m = jnp.min(d, axis=0, keepdims=True)
    K = d.shape[0]
    j = lax.broadcasted_iota(jnp.int32, d.shape, 0)
    idx = jnp.min(jnp.where(d == m, j, K), axis=0).astype(jnp.int32)
    idx_ref[0, 0, :] = idx


def _compute_indices(zf, codebook, block_m):
    M, D = zf.shape
    K = codebook.shape[0]
    nb = M // block_m
    z2 = jnp.sum(zf * zf, axis=-1, keepdims=True).reshape(nb, 1, block_m)
    c2 = jnp.sum(codebook * codebook, axis=-1).reshape(K, 1)
    idx3 = pl.pallas_call(
        _argmin_body,
        grid=(nb,),
        in_specs=[
            pl.BlockSpec((block_m, D), lambda i: (i, 0)),
            pl.BlockSpec(memory_space=pl.ANY),
            pl.BlockSpec((1, 1, block_m), lambda i: (i, 0, 0)),
            pl.BlockSpec((K, 1), lambda i: (0, 0)),
        ],
        out_specs=pl.BlockSpec((1, 1, block_m), lambda i: (i, 0, 0)),
        out_shape=jax.ShapeDtypeStruct((nb, 1, block_m), jnp.int32),
        scratch_shapes=[
            pltpu.VMEM((K, D), jnp.float32),
            pltpu.SemaphoreType.DMA,
        ],
    )(zf, codebook, z2, c2)
    return idx3.reshape(M)


def _gather_rows(idx, codebook, n_rows, d):
    info = plsc.get_sparse_core_info()
    nc, ns = info.num_cores, info.num_subcores
    nw = nc * ns
    b_per_w = n_rows // nw
    ch = 32
    nch = b_per_w // ch
    mesh = plsc.VectorSubcoreMesh(core_axis_name="c", subcore_axis_name="s")

    @functools.partial(
        pl.kernel,
        mesh=mesh,
        out_type=jax.ShapeDtypeStruct((n_rows, d), jnp.float32),
        scratch_types=[
            pltpu.VMEM((b_per_w,), jnp.int32),
            pltpu.VMEM((ch, d), jnp.float32),
            pltpu.VMEM((ch, d), jnp.float32),
            pltpu.SemaphoreType.DMA,
            pltpu.SemaphoreType.DMA,
            pltpu.SemaphoreType.DMA,
            pltpu.SemaphoreType.DMA,
        ],
    )
    def gather_kernel(idx_hbm, cb_hbm, out_hbm, idx_v, buf0, buf1,
                      gs0, gs1, ws0, ws1):
        wid = lax.axis_index("s") * nc + lax.axis_index("c")
        base = wid * b_per_w
        bufs = (buf0, buf1)
        gsems = (gs0, gs1)
        wsems = (ws0, ws1)
        pltpu.sync_copy(idx_hbm.at[pl.ds(base, b_per_w)], idx_v)

        gathers = [pltpu.async_copy(
            cb_hbm.at[idx_v.at[pl.ds(0, ch)]], bufs[0], gsems[0])]
        writes = []
        for c in range(nch):
            if c + 1 < nch:
                if c >= 1:
                    writes[c - 1].wait()
                gathers.append(pltpu.async_copy(
                    cb_hbm.at[idx_v.at[pl.ds((c + 1) * ch, ch)]],
                    bufs[(c + 1) % 2], gsems[(c + 1) % 2]))
            gathers[c].wait()
            writes.append(pltpu.async_copy(
                bufs[c % 2], out_hbm.at[pl.ds(base + c * ch, ch)],
                wsems[c % 2]))
        writes[nch - 2].wait()
        writes[nch - 1].wait()

    return gather_kernel(idx, codebook)


def kernel(z, codebook):
    b, t, d = z.shape
    zf = z.reshape(b * t, d)
    idx = _compute_indices(zf, codebook, block_m=512)
    quant = _gather_rows(idx, codebook, b * t, d)
    return quant.reshape(b, t, d), idx.reshape(b, t)


# speedup vs baseline: 1.3619x; 1.0288x over previous
import functools

import jax
import jax.numpy as jnp
from jax import lax
from jax.experimental import pallas as pl
from jax.experimental.pallas import tpu as pltpu
from jax.experimental.pallas import tpu_sc as plsc


def _argmin_body(z_ref, cb_hbm, z2_ref, c2_ref, idx_ref, cb_v, sem):
    @pl.when(pl.program_id(0) == 0)
    def _stage():
        cp = pltpu.make_async_copy(cb_hbm, cb_v, sem)
        cp.start()
        cp.wait()

    zb = z_ref[...]
    cb = cb_v[...]
    z2 = z2_ref[...].reshape(1, -1)
    c2 = c2_ref[...].reshape(-1, 1)
    s = lax.dot_general(cb, zb, (((1,), (1,)), ((), ())),
                        preferred_element_type=jnp.float32)
    sq = (z2 + c2) - 2.0 * s
    d = jnp.sqrt(jnp.maximum(sq, 0.0))
    m = jnp.min(d, axis=0, keepdims=True)
    K = d.shape[0]
    j = lax.broadcasted_iota(jnp.int32, d.shape, 0)
    idx = jnp.min(jnp.where(d == m, j, K), axis=0).astype(jnp.int32)
    idx_ref[0, 0, :] = idx


def _compute_indices(zf, codebook, block_m):
    M, D = zf.shape
    K = codebook.shape[0]
    nb = M // block_m
    z2 = jnp.sum(zf * zf, axis=-1, keepdims=True).reshape(nb, 1, block_m)
    c2 = jnp.sum(codebook * codebook, axis=-1).reshape(K, 1)
    idx3 = pl.pallas_call(
        _argmin_body,
        grid=(nb,),
        in_specs=[
            pl.BlockSpec((block_m, D), lambda i: (i, 0)),
            pl.BlockSpec(memory_space=pl.ANY),
            pl.BlockSpec((1, 1, block_m), lambda i: (i, 0, 0)),
            pl.BlockSpec((K, 1), lambda i: (0, 0)),
        ],
        out_specs=pl.BlockSpec((1, 1, block_m), lambda i: (i, 0, 0)),
        out_shape=jax.ShapeDtypeStruct((nb, 1, block_m), jnp.int32),
        scratch_shapes=[
            pltpu.VMEM((K, D), jnp.float32),
            pltpu.SemaphoreType.DMA,
        ],
    )(zf, codebook, z2, c2)
    return idx3.reshape(M)


def _gather_rows(idx, codebook, n_rows, d):
    info = plsc.get_sparse_core_info()
    nc, ns = info.num_cores, info.num_subcores
    nw = nc * ns
    b_per_w = n_rows // nw
    ch = 32
    nch = b_per_w // ch
    mesh = plsc.VectorSubcoreMesh(core_axis_name="c", subcore_axis_name="s")

    @functools.partial(
        pl.kernel,
        mesh=mesh,
        out_type=jax.ShapeDtypeStruct((n_rows, d), jnp.float32),
        scratch_types=[
            pltpu.VMEM((b_per_w,), jnp.int32),
            pltpu.VMEM((ch, d), jnp.float32),
            pltpu.VMEM((ch, d), jnp.float32),
            pltpu.SemaphoreType.DMA,
            pltpu.SemaphoreType.DMA,
            pltpu.SemaphoreType.DMA,
            pltpu.SemaphoreType.DMA,
        ],
    )
    def gather_kernel(idx_hbm, cb_hbm, out_hbm, idx_v, buf0, buf1,
                      gs0, gs1, ws0, ws1):
        wid = lax.axis_index("s") * nc + lax.axis_index("c")
        base = wid * b_per_w
        bufs = (buf0, buf1)
        gsems = (gs0, gs1)
        wsems = (ws0, ws1)
        pltpu.sync_copy(idx_hbm.at[pl.ds(base, b_per_w)], idx_v)

        gathers = [pltpu.async_copy(
            cb_hbm.at[idx_v.at[pl.ds(0, ch)]], bufs[0], gsems[0])]
        writes = []
        for c in range(nch):
            if c + 1 < nch:
                if c >= 1:
                    writes[c - 1].wait()
                gathers.append(pltpu.async_copy(
                    cb_hbm.at[idx_v.at[pl.ds((c + 1) * ch, ch)]],
                    bufs[(c + 1) % 2], gsems[(c + 1) % 2]))
            gathers[c].wait()
            writes.append(pltpu.async_copy(
                bufs[c % 2], out_hbm.at[pl.ds(base + c * ch, ch)],
                wsems[c % 2]))
        writes[nch - 2].wait()
        writes[nch - 1].wait()

    return gather_kernel(idx, codebook)


def kernel(z, codebook):
    b, t, d = z.shape
    zf = z.reshape(b * t, d)
    idx = _compute_indices(zf, codebook, block_m=1024)
    quant = _gather_rows(idx, codebook, b * t, d)
    return quant.reshape(b, t, d), idx.reshape(b, t)
